# Initial kernel scaffold; baseline (speedup 1.0000x reference)
import functools

import jax
import jax.numpy as jnp
from jax import lax
from jax.experimental import pallas as pl
from jax.experimental.pallas import tpu as pltpu
from jax.experimental.pallas import tpu_sc as plsc

N_REGIONS = 100000
N_EMB = 16
N_INTER = 16
N_CELLS = 256
N_REG_B = 4096

NC = 2
NS = 16
NW = NC * NS
BPW = N_REG_B // NW

_SC_MESH = plsc.VectorSubcoreMesh(core_axis_name="c", subcore_axis_name="s")


@functools.partial(
    pl.kernel,
    mesh=_SC_MESH,
    out_type=(
        jax.ShapeDtypeStruct((N_REG_B, N_EMB * N_INTER), jnp.float32),
        jax.ShapeDtypeStruct((N_REG_B, N_INTER), jnp.float32),
        jax.ShapeDtypeStruct((N_REG_B, N_INTER), jnp.float32),
        jax.ShapeDtypeStruct((N_REG_B,), jnp.float32),
    ),
    scratch_types=(
        pltpu.VMEM((BPW,), jnp.int32),
        pltpu.VMEM((BPW,), jnp.int32),
        pltpu.VMEM((BPW, N_EMB * N_INTER), jnp.float32),
        pltpu.VMEM((BPW, N_INTER), jnp.float32),
        pltpu.VMEM((BPW, N_INTER), jnp.float32),
        pltpu.VMEM((BPW, 16), jnp.float32),
        pltpu.VMEM((BPW,), jnp.float32),
        pltpu.SemaphoreType.DMA,
    ),
)
def _sc_gather(w1_hbm, b1_hbm, w2_hbm, b2_hbm, ix_hbm,
               w1o, b1o, w2o, b2o,
               idx_v, idxhi_v, w1v, b1v, w2v, b2rows_v, b2v, sem):
    wid = lax.axis_index("s") * NC + lax.axis_index("c")
    base = wid * BPW
    pltpu.sync_copy(ix_hbm.at[pl.ds(base, BPW)], idx_v)

    for j in range(BPW // 16):
        v = idx_v[pl.ds(j * 16, 16)]
        idxhi_v[pl.ds(j * 16, 16)] = lax.shift_right_logical(v, 4)

    pltpu.async_copy(w1_hbm.at[idx_v], w1v, sem).wait()
    pltpu.async_copy(b1_hbm.at[idx_v], b1v, sem).wait()
    pltpu.async_copy(w2_hbm.at[idx_v], w2v, sem).wait()
    pltpu.async_copy(b2_hbm.at[idxhi_v], b2rows_v, sem).wait()

    for j in range(BPW // 16):
        rows = lax.iota(jnp.int32, (16,)) + j * 16
        lanes = idx_v[pl.ds(j * 16, 16)] & 15
        b2v[pl.ds(j * 16, 16)] = plsc.load_gather(b2rows_v, [rows, lanes])

    pltpu.sync_copy(w1v, w1o.at[pl.ds(base, BPW)])
    pltpu.sync_copy(b1v, b1o.at[pl.ds(base, BPW)])
    pltpu.sync_copy(w2v, w2o.at[pl.ds(base, BPW)])
    pltpu.sync_copy(b2v, b2o.at[pl.ds(base, BPW)])


B_R = 512
C_CH = 32


def _tc_body(xt_ref, w1_ref, b1_ref, w2_ref, b2_ref, out_ref):
    def chunk(i, carry):
        a0 = pl.multiple_of(i * C_CH, C_CH)
        acc = jnp.broadcast_to(b2_ref[0:1, :], (C_CH, B_R))
        for d in range(N_INTER):
            hd = jnp.broadcast_to(b1_ref[d:d + 1, :], (C_CH, B_R))
            for c in range(N_EMB):
                hd = hd + xt_ref[c, pl.ds(a0, C_CH), :] * w1_ref[16 * c + d:16 * c + d + 1, :]
            hs = 0.5 * jnp.tanh(0.5 * hd) + 0.5
            acc = acc + hs * w2_ref[d:d + 1, :]
        out_ref[pl.ds(a0, C_CH), :] = acc
        return carry

    lax.fori_loop(0, N_CELLS // C_CH, chunk, 0)


def _tc_compute(xt, w1cd, b1t, w2t, b2row):
    nb = N_REG_B // B_R
    return pl.pallas_call(
        _tc_body,
        grid=(nb,),
        in_specs=[
            pl.BlockSpec((N_EMB, N_CELLS, B_R), lambda i: (0, 0, i)),
            pl.BlockSpec((N_EMB * N_INTER, B_R), lambda i: (0, i)),
            pl.BlockSpec((N_INTER, B_R), lambda i: (0, i)),
            pl.BlockSpec((N_INTER, B_R), lambda i: (0, i)),
            pl.BlockSpec((1, B_R), lambda i: (0, i)),
        ],
        out_specs=pl.BlockSpec((N_CELLS, B_R), lambda i: (0, i)),
        out_shape=jax.ShapeDtypeStruct((N_CELLS, N_REG_B), jnp.float32),
    )(xt, w1cd, b1t, w2t, b2row)


def kernel(cell_region_embedding, region_ix, W1, b1, W2, b2):
    ix = region_ix.astype(jnp.int32)
    w1r = W1.reshape(N_REGIONS, N_EMB * N_INTER)
    w2r = W2.reshape(N_REGIONS, N_INTER)
    b2r = b2.reshape(N_REGIONS // 16, 16)

    w1g, b1g, w2g, b2g = _sc_gather(w1r, b1, w2r, b2r, ix)

    xt = jnp.transpose(cell_region_embedding, (2, 0, 1))
    w1cd = jnp.transpose(w1g.reshape(N_REG_B, N_EMB, N_INTER), (1, 2, 0))
    w1cd = w1cd.reshape(N_EMB * N_INTER, N_REG_B)
    b1t = b1g.T
    w2t = w2g.T
    b2row = b2g.reshape(1, N_REG_B)

    return _tc_compute(xt, w1cd, b1t, w2t, b2row)


# speedup vs baseline: 1.3221x; 1.3221x over previous
import functools

import jax
import jax.numpy as jnp
from jax import lax
from jax.experimental import pallas as pl
from jax.experimental.pallas import tpu as pltpu
from jax.experimental.pallas import tpu_sc as plsc

N_REGIONS = 100000
N_EMB = 16
N_INTER = 16
N_CELLS = 256
N_REG_B = 4096

NC = 2
NS = 16
NW = NC * NS
BPW = N_REG_B // NW

@functools.cache
def _make_sc_gather():
  mesh = plsc.VectorSubcoreMesh(core_axis_name="c", subcore_axis_name="s")

  @functools.partial(
    pl.kernel,
    mesh=mesh,
    out_type=(
        jax.ShapeDtypeStruct((N_REG_B, N_EMB * N_INTER), jnp.float32),
        jax.ShapeDtypeStruct((N_REG_B, N_INTER), jnp.float32),
        jax.ShapeDtypeStruct((N_REG_B, N_INTER), jnp.float32),
        jax.ShapeDtypeStruct((N_REG_B, 16), jnp.float32),
    ),
    scratch_types=(
        pltpu.VMEM((BPW,), jnp.int32),
        pltpu.VMEM((BPW,), jnp.int32),
        pltpu.VMEM((BPW, N_EMB * N_INTER), jnp.float32),
        pltpu.VMEM((BPW, N_INTER), jnp.float32),
        pltpu.VMEM((BPW, N_INTER), jnp.float32),
        pltpu.VMEM((BPW, 16), jnp.float32),
        pltpu.SemaphoreType.DMA,
    ),
    compiler_params=pltpu.CompilerParams(use_tc_tiling_on_sc=False),
  )
  def _sc_gather(w1_hbm, b1_hbm, w2_hbm, b2_hbm, ix_hbm,
                 w1o, b1o, w2o, b2o,
                 idx_v, idxhi_v, w1v, b1v, w2v, b2v, sem):
    wid = lax.axis_index("s") * NC + lax.axis_index("c")
    base = wid * BPW
    pltpu.sync_copy(ix_hbm.at[pl.ds(base, BPW)], idx_v)

    for j in range(BPW // 16):
        v = idx_v[pl.ds(j * 16, 16)]
        idxhi_v[pl.ds(j * 16, 16)] = lax.shift_right_logical(v, 4)

    pltpu.async_copy(w1_hbm.at[idx_v], w1v, sem).wait()
    pltpu.async_copy(b1_hbm.at[idx_v], b1v, sem).wait()
    pltpu.async_copy(w2_hbm.at[idx_v], w2v, sem).wait()
    pltpu.async_copy(b2_hbm.at[idxhi_v], b2v, sem).wait()

    pltpu.sync_copy(w1v, w1o.at[pl.ds(base, BPW)])
    pltpu.sync_copy(b1v, b1o.at[pl.ds(base, BPW)])
    pltpu.sync_copy(w2v, w2o.at[pl.ds(base, BPW)])
    pltpu.sync_copy(b2v, b2o.at[pl.ds(base, BPW)])

  return _sc_gather


B_R = 512
C_CH = 32


def _tc_body(xt_ref, w1_ref, b1_ref, w2_ref, b2t_ref, lo_ref, out_ref):
    lo = lo_ref[0:1, :]
    b2row = jnp.zeros((1, B_R), jnp.float32)
    for l in range(16):
        b2row = b2row + jnp.where(lo == l, b2t_ref[l:l + 1, :], 0.0)

    def chunk(i, carry):
        a0 = pl.multiple_of(i * C_CH, C_CH)
        acc = jnp.broadcast_to(b2row, (C_CH, B_R))
        for d in range(N_INTER):
            hd = jnp.broadcast_to(b1_ref[d:d + 1, :], (C_CH, B_R))
            for c in range(N_EMB):
                hd = hd + xt_ref[c, pl.ds(a0, C_CH), :] * w1_ref[16 * c + d:16 * c + d + 1, :]
            hs = 0.5 * jnp.tanh(0.5 * hd) + 0.5
            acc = acc + hs * w2_ref[d:d + 1, :]
        out_ref[pl.ds(a0, C_CH), :] = acc
        return carry

    lax.fori_loop(0, N_CELLS // C_CH, chunk, 0)


def _tc_compute(xt, w1cd, b1t, w2t, b2t, lorow):
    nb = N_REG_B // B_R
    return pl.pallas_call(
        _tc_body,
        grid=(nb,),
        in_specs=[
            pl.BlockSpec((N_EMB, N_CELLS, B_R), lambda i: (0, 0, i)),
            pl.BlockSpec((N_EMB * N_INTER, B_R), lambda i: (0, i)),
            pl.BlockSpec((N_INTER, B_R), lambda i: (0, i)),
            pl.BlockSpec((N_INTER, B_R), lambda i: (0, i)),
            pl.BlockSpec((16, B_R), lambda i: (0, i)),
            pl.BlockSpec((1, B_R), lambda i: (0, i)),
        ],
        out_specs=pl.BlockSpec((N_CELLS, B_R), lambda i: (0, i)),
        out_shape=jax.ShapeDtypeStruct((N_CELLS, N_REG_B), jnp.float32),
    )(xt, w1cd, b1t, w2t, b2t, lorow)


def kernel(cell_region_embedding, region_ix, W1, b1, W2, b2):
    ix = region_ix.astype(jnp.int32)
    w1r = W1.reshape(N_REGIONS, N_EMB * N_INTER)
    w2r = W2.reshape(N_REGIONS, N_INTER)

    b2r = b2.reshape(N_REGIONS // 16, 16)
    w1g, b1g, w2g, b2g = _make_sc_gather()(w1r, b1, w2r, b2r, ix)

    xt = jnp.transpose(cell_region_embedding, (2, 0, 1))
    w1cd = jnp.transpose(w1g.reshape(N_REG_B, N_EMB, N_INTER), (1, 2, 0))
    w1cd = w1cd.reshape(N_EMB * N_INTER, N_REG_B)
    b1t = b1g.T
    w2t = w2g.T
    b2t = b2g.T
    lorow = (ix & 15).reshape(1, N_REG_B)

    return _tc_compute(xt, w1cd, b1t, w2t, b2t, lorow)
